# baseline (speedup 1.0000x reference)
import functools

import jax
import jax.numpy as jnp
from jax import lax
from jax.experimental import pallas as pl
from jax.experimental.pallas import tpu as pltpu
from jax.experimental.pallas import tpu_sc as plsc


def _routing(topk_ids, topk_weights, E, BLK_M, G):
    T = topk_ids.shape[0]
    GM = G * BLK_M
    eid = topk_ids[:, 0].astype(jnp.int32)
    w = topk_weights[:, 0].astype(jnp.float32)
    C = 16
    L = T // C
    oh = (eid[:, None] == jnp.arange(E, dtype=jnp.int32)[None, :]).astype(
        jnp.float32
    )
    oh3 = oh.reshape(C, L, E)
    tril = jnp.tril(jnp.ones((L, L), jnp.float32))
    local = jnp.matmul(tril, oh3)
    chunk_tot = oh3.sum(axis=1)
    offs = jnp.cumsum(chunk_tot, axis=0) - chunk_tot
    cum = local + offs[:, None, :]
    counts = chunk_tot.sum(axis=0).astype(jnp.int32)
    ranks = (cum * oh3).sum(axis=-1).reshape(T).astype(jnp.int32) - 1
    blocks_e = (counts + BLK_M - 1) // BLK_M
    blk_end = jnp.cumsum(blocks_e)
    blk_start = blk_end - blocks_e
    base = (oh * blk_start[None, :].astype(jnp.float32)).sum(-1).astype(
        jnp.int32
    )
    pos = base * BLK_M + ranks
    block_expert = jnp.minimum(
        (blk_end[None, :] <= jnp.arange(G, dtype=jnp.int32)[:, None])
        .astype(jnp.int32).sum(axis=1),
        E - 1,
    ).astype(jnp.int32)
    return pos, w, block_expert


def _mm_body(be_ref, xs_ref, w0_ref, w1_ref, w2_ref, w3_ref, pw_ref, ys_ref):
    xb = xs_ref[...]
    acc = None
    for j, wj in enumerate((w0_ref, w1_ref, w2_ref, w3_ref)):
        kj = wj.shape[-2]
        part = jnp.dot(xb[:, j * kj:(j + 1) * kj], wj[0, 0],
                       preferred_element_type=jnp.float32)
        acc = part if acc is None else acc + part
    ys_ref[...] = acc * pw_ref[0]


def kernel(x, topk_weights, topk_ids, W1):
    T, K = x.shape
    E, _, N = W1.shape
    BLK_M = 128
    G = T // BLK_M + E
    GM = G * BLK_M

    pos, wtok, block_expert = _routing(topk_ids, topk_weights, E, BLK_M, G)

    info = plsc.get_sparse_core_info()
    NC, NS = info.num_cores, info.num_subcores
    NW = NC * NS
    CH = T // NW
    mesh = plsc.VectorSubcoreMesh(core_axis_name="c", subcore_axis_name="s")

    @functools.partial(
        pl.kernel,
        out_type=(
            jax.ShapeDtypeStruct((GM, K), jnp.float32),
            jax.ShapeDtypeStruct((GM,), jnp.float32),
        ),
        mesh=mesh,
        scratch_types=[
            pltpu.VMEM((CH,), jnp.int32),
            pltpu.VMEM((CH,), jnp.float32),
            pltpu.VMEM((CH, K), jnp.float32),
            pltpu.SemaphoreType.DMA,
        ],
    )
    def dispatch(x_hbm, w_hbm, pos_hbm, xs_hbm, padw_hbm, idx_v, w_v, rows_v,
                 sem):
        wid = lax.axis_index("s") * NC + lax.axis_index("c")
        base = wid * CH
        pltpu.sync_copy(pos_hbm.at[pl.ds(base, CH)], idx_v)
        pltpu.sync_copy(w_hbm.at[pl.ds(base, CH)], w_v)
        pltpu.sync_copy(x_hbm.at[pl.ds(base, CH)], rows_v)
        pltpu.async_copy(rows_v, xs_hbm.at[idx_v], sem).wait()
        pltpu.async_copy(w_v, padw_hbm.at[idx_v], sem).wait()

    @functools.partial(
        pl.kernel,
        out_type=jax.ShapeDtypeStruct((T, N), jnp.float32),
        mesh=mesh,
        scratch_types=[
            pltpu.VMEM((CH,), jnp.int32),
            pltpu.VMEM((CH, N), jnp.float32),
            pltpu.SemaphoreType.DMA,
        ],
    )
    def combine(ys_hbm, pos_hbm, out_hbm, idx_v, rows_v, sem):
        wid = lax.axis_index("s") * NC + lax.axis_index("c")
        base = wid * CH
        pltpu.sync_copy(pos_hbm.at[pl.ds(base, CH)], idx_v)
        pltpu.async_copy(ys_hbm.at[idx_v], rows_v, sem).wait()
        pltpu.sync_copy(rows_v, out_hbm.at[pl.ds(base, CH)])

    xs, pad_w = dispatch(x, wtok, pos)

    S = 4
    KS_ = K // S
    W1r = W1.reshape(E, S, KS_, N)

    def _w_spec(j):
        return pl.BlockSpec((1, 1, KS_, N), lambda b, be, j=j: (be[b], j, 0, 0))

    grid_spec = pltpu.PrefetchScalarGridSpec(
        num_scalar_prefetch=1,
        grid=(G,),
        in_specs=[
            pl.BlockSpec((BLK_M, K), lambda b, be: (b, 0)),
            _w_spec(0), _w_spec(1), _w_spec(2), _w_spec(3),
            pl.BlockSpec((1, BLK_M, 1), lambda b, be: (b, 0, 0)),
        ],
        out_specs=pl.BlockSpec((BLK_M, N), lambda b, be: (b, 0)),
    )
    ys = pl.pallas_call(
        _mm_body,
        grid_spec=grid_spec,
        out_shape=jax.ShapeDtypeStruct((GM, N), jnp.float32),
        compiler_params=pltpu.CompilerParams(
            dimension_semantics=("arbitrary",),
        ),
    )(block_expert, xs, W1r, W1r, W1r, W1r, pad_w.reshape(G, BLK_M, 1))

    return combine(ys, pos)


# speedup vs baseline: 1.0047x; 1.0047x over previous
import functools

import jax
import jax.numpy as jnp
from jax import lax
from jax.experimental import pallas as pl
from jax.experimental.pallas import tpu as pltpu
from jax.experimental.pallas import tpu_sc as plsc


def _routing(topk_ids, topk_weights, E, BLK_M, G):
    T = topk_ids.shape[0]
    GM = G * BLK_M
    eid = topk_ids[:, 0].astype(jnp.int32)
    w = topk_weights[:, 0].astype(jnp.float32)
    C = 16
    L = T // C
    oh = (eid[:, None] == jnp.arange(E, dtype=jnp.int32)[None, :]).astype(
        jnp.float32
    )
    oh3 = oh.reshape(C, L, E)
    tril = jnp.tril(jnp.ones((L, L), jnp.float32))
    local = jnp.matmul(tril, oh3)
    chunk_tot = oh3.sum(axis=1)
    offs = jnp.cumsum(chunk_tot, axis=0) - chunk_tot
    cum = local + offs[:, None, :]
    counts = chunk_tot.sum(axis=0).astype(jnp.int32)
    ranks = (cum * oh3).sum(axis=-1).reshape(T).astype(jnp.int32) - 1
    blocks_e = (counts + BLK_M - 1) // BLK_M
    blk_end = jnp.cumsum(blocks_e)
    blk_start = blk_end - blocks_e
    base = (oh * blk_start[None, :].astype(jnp.float32)).sum(-1).astype(
        jnp.int32
    )
    pos = base * BLK_M + ranks
    block_expert = jnp.minimum(
        (blk_end[None, :] <= jnp.arange(G, dtype=jnp.int32)[:, None])
        .astype(jnp.int32).sum(axis=1),
        E - 1,
    ).astype(jnp.int32)
    return pos, w, block_expert


def _mm_body(be_ref, xs_ref, w0_ref, w1_ref, w2_ref, w3_ref, pw_ref, ys_ref):
    xb = xs_ref[...]
    acc = None
    for j, wj in enumerate((w0_ref, w1_ref, w2_ref, w3_ref)):
        kj = wj.shape[-2]
        part = jnp.dot(xb[:, j * kj:(j + 1) * kj], wj[0, 0],
                       preferred_element_type=jnp.float32)
        acc = part if acc is None else acc + part
    ys_ref[...] = acc * pw_ref[0]


def kernel(x, topk_weights, topk_ids, W1):
    T, K = x.shape
    E, _, N = W1.shape
    BLK_M = 128
    G = T // BLK_M + E
    GM = G * BLK_M

    pos, wtok, block_expert = _routing(topk_ids, topk_weights, E, BLK_M, G)

    info = plsc.get_sparse_core_info()
    NC, NS = info.num_cores, info.num_subcores
    NW = NC * NS
    CH = T // NW
    mesh = plsc.VectorSubcoreMesh(core_axis_name="c", subcore_axis_name="s")

    @functools.partial(
        pl.kernel,
        out_type=(
            jax.ShapeDtypeStruct((GM, K), jnp.float32),
            jax.ShapeDtypeStruct((GM,), jnp.float32),
        ),
        mesh=mesh,
        scratch_types=[
            pltpu.VMEM((CH,), jnp.int32),
            pltpu.VMEM((CH,), jnp.float32),
            pltpu.VMEM((CH, K), jnp.float32),
            pltpu.SemaphoreType.DMA,
            pltpu.SemaphoreType.DMA,
        ],
    )
    def dispatch(x_hbm, w_hbm, pos_hbm, xs_hbm, padw_hbm, idx_v, w_v, rows_v,
                 sem, sem2):
        wid = lax.axis_index("s") * NC + lax.axis_index("c")
        base = wid * CH
        pltpu.sync_copy(pos_hbm.at[pl.ds(base, CH)], idx_v)
        pltpu.sync_copy(w_hbm.at[pl.ds(base, CH)], w_v)
        pltpu.sync_copy(x_hbm.at[pl.ds(base, CH)], rows_v)
        cw = pltpu.async_copy(w_v, padw_hbm.at[idx_v], sem2)
        cx = pltpu.async_copy(rows_v, xs_hbm.at[idx_v], sem)
        cx.wait()
        cw.wait()

    @functools.partial(
        pl.kernel,
        out_type=jax.ShapeDtypeStruct((T, N), jnp.float32),
        mesh=mesh,
        scratch_types=[
            pltpu.VMEM((CH,), jnp.int32),
            pltpu.VMEM((CH, N), jnp.float32),
            pltpu.SemaphoreType.DMA,
        ],
    )
    def combine(ys_hbm, pos_hbm, out_hbm, idx_v, rows_v, sem):
        wid = lax.axis_index("s") * NC + lax.axis_index("c")
        base = wid * CH
        pltpu.sync_copy(pos_hbm.at[pl.ds(base, CH)], idx_v)
        pltpu.async_copy(ys_hbm.at[idx_v], rows_v, sem).wait()
        pltpu.sync_copy(rows_v, out_hbm.at[pl.ds(base, CH)])

    xs, pad_w = dispatch(x, wtok, pos)

    S = 4
    KS_ = K // S
    W1r = W1.reshape(E, S, KS_, N)

    def _w_spec(j):
        return pl.BlockSpec((1, 1, KS_, N), lambda b, be, j=j: (be[b], j, 0, 0))

    grid_spec = pltpu.PrefetchScalarGridSpec(
        num_scalar_prefetch=1,
        grid=(G,),
        in_specs=[
            pl.BlockSpec((BLK_M, K), lambda b, be: (b, 0)),
            _w_spec(0), _w_spec(1), _w_spec(2), _w_spec(3),
            pl.BlockSpec((1, BLK_M, 1), lambda b, be: (b, 0, 0)),
        ],
        out_specs=pl.BlockSpec((BLK_M, N), lambda b, be: (b, 0)),
    )
    ys = pl.pallas_call(
        _mm_body,
        grid_spec=grid_spec,
        out_shape=jax.ShapeDtypeStruct((GM, N), jnp.float32),
        compiler_params=pltpu.CompilerParams(
            dimension_semantics=("arbitrary",),
        ),
    )(block_expert, xs, W1r, W1r, W1r, W1r, pad_w.reshape(G, BLK_M, 1))

    return combine(ys, pos)


# speedup vs baseline: 1.0490x; 1.0441x over previous
import functools

import jax
import jax.numpy as jnp
from jax import lax
from jax.experimental import pallas as pl
from jax.experimental.pallas import tpu as pltpu
from jax.experimental.pallas import tpu_sc as plsc


def _routing(topk_ids, topk_weights, E, BLK_M, G):
    T = topk_ids.shape[0]
    GM = G * BLK_M
    eid = topk_ids[:, 0].astype(jnp.int32)
    w = topk_weights[:, 0].astype(jnp.float32)
    C = 16
    L = T // C
    oh = (eid[:, None] == jnp.arange(E, dtype=jnp.int32)[None, :]).astype(
        jnp.float32
    )
    oh3 = oh.reshape(C, L, E)
    tril = jnp.tril(jnp.ones((L, L), jnp.float32))
    local = jnp.matmul(tril, oh3)
    chunk_tot = oh3.sum(axis=1)
    offs = jnp.cumsum(chunk_tot, axis=0) - chunk_tot
    cum = local + offs[:, None, :]
    counts = chunk_tot.sum(axis=0).astype(jnp.int32)
    ranks = (cum * oh3).sum(axis=-1).reshape(T).astype(jnp.int32) - 1
    blocks_e = (counts + BLK_M - 1) // BLK_M
    blk_end = jnp.cumsum(blocks_e)
    blk_start = blk_end - blocks_e
    base = (oh * blk_start[None, :].astype(jnp.float32)).sum(-1).astype(
        jnp.int32
    )
    pos = base * BLK_M + ranks
    pad_w = jnp.zeros((GM,), jnp.float32).at[pos].set(w)
    block_expert = jnp.minimum(
        (blk_end[None, :] <= jnp.arange(G, dtype=jnp.int32)[:, None])
        .astype(jnp.int32).sum(axis=1),
        E - 1,
    ).astype(jnp.int32)
    return pos, pad_w, block_expert


def _mm_body(be_ref, xs_ref, w0_ref, w1_ref, w2_ref, w3_ref, pw_ref, ys_ref):
    xb = xs_ref[...]
    acc = None
    for j, wj in enumerate((w0_ref, w1_ref, w2_ref, w3_ref)):
        kj = wj.shape[-2]
        part = jnp.dot(xb[:, j * kj:(j + 1) * kj], wj[0, 0],
                       preferred_element_type=jnp.float32)
        acc = part if acc is None else acc + part
    ys_ref[...] = acc * pw_ref[0]


def kernel(x, topk_weights, topk_ids, W1):
    T, K = x.shape
    E, _, N = W1.shape
    BLK_M = 128
    G = T // BLK_M + E
    GM = G * BLK_M

    pos, pad_w, block_expert = _routing(topk_ids, topk_weights, E, BLK_M, G)

    info = plsc.get_sparse_core_info()
    NC, NS = info.num_cores, info.num_subcores
    NW = NC * NS
    CH = T // NW
    mesh = plsc.VectorSubcoreMesh(core_axis_name="c", subcore_axis_name="s")

    @functools.partial(
        pl.kernel,
        out_type=jax.ShapeDtypeStruct((GM, K), jnp.float32),
        mesh=mesh,
        scratch_types=[
            pltpu.VMEM((CH,), jnp.int32),
            pltpu.VMEM((CH, K), jnp.float32),
            pltpu.SemaphoreType.DMA,
        ],
    )
    def dispatch(x_hbm, pos_hbm, xs_hbm, idx_v, rows_v, sem):
        wid = lax.axis_index("s") * NC + lax.axis_index("c")
        base = wid * CH
        pltpu.sync_copy(pos_hbm.at[pl.ds(base, CH)], idx_v)
        pltpu.sync_copy(x_hbm.at[pl.ds(base, CH)], rows_v)
        pltpu.async_copy(rows_v, xs_hbm.at[idx_v], sem).wait()

    @functools.partial(
        pl.kernel,
        out_type=jax.ShapeDtypeStruct((T, N), jnp.float32),
        mesh=mesh,
        scratch_types=[
            pltpu.VMEM((CH,), jnp.int32),
            pltpu.VMEM((CH, N), jnp.float32),
            pltpu.SemaphoreType.DMA,
        ],
    )
    def combine(ys_hbm, pos_hbm, out_hbm, idx_v, rows_v, sem):
        wid = lax.axis_index("s") * NC + lax.axis_index("c")
        base = wid * CH
        pltpu.sync_copy(pos_hbm.at[pl.ds(base, CH)], idx_v)
        pltpu.async_copy(ys_hbm.at[idx_v], rows_v, sem).wait()
        pltpu.sync_copy(rows_v, out_hbm.at[pl.ds(base, CH)])

    xs = dispatch(x, pos)

    S = 4
    KS_ = K // S
    W1r = W1.reshape(E, S, KS_, N)

    def _w_spec(j):
        return pl.BlockSpec((1, 1, KS_, N), lambda b, be, j=j: (be[b], j, 0, 0))

    grid_spec = pltpu.PrefetchScalarGridSpec(
        num_scalar_prefetch=1,
        grid=(G,),
        in_specs=[
            pl.BlockSpec((BLK_M, K), lambda b, be: (b, 0)),
            _w_spec(0), _w_spec(1), _w_spec(2), _w_spec(3),
            pl.BlockSpec((1, BLK_M, 1), lambda b, be: (b, 0, 0)),
        ],
        out_specs=pl.BlockSpec((BLK_M, N), lambda b, be: (b, 0)),
    )
    ys = pl.pallas_call(
        _mm_body,
        grid_spec=grid_spec,
        out_shape=jax.ShapeDtypeStruct((GM, N), jnp.float32),
        compiler_params=pltpu.CompilerParams(
            dimension_semantics=("arbitrary",),
        ),
    )(block_expert, xs, W1r, W1r, W1r, W1r, pad_w.reshape(G, BLK_M, 1))

    return combine(ys, pos)
